# baseline (speedup 1.0000x reference)
import functools

import jax
import jax.numpy as jnp
from jax import lax
from jax.experimental import pallas as pl
from jax.experimental.pallas import tpu as pltpu
from jax.experimental.pallas import tpu_sc as plsc

NC = 2
NS = 16
NW = NC * NS
L = 16

B_TOTAL = 1024 * 26 * 20
D = 32
B_PER_W = B_TOTAL // NW
CHUNK = 128
NCHUNK = B_PER_W // CHUNK
NGRP = B_PER_W // L
OUT_CH = CHUNK // 4


@functools.partial(
    pl.kernel,
    out_type=jax.ShapeDtypeStruct((B_TOTAL // 4, 4 * D), jnp.float32),
    mesh=plsc.VectorSubcoreMesh(core_axis_name="c", subcore_axis_name="s"),
    scratch_types=[
        pltpu.VMEM((B_PER_W,), jnp.int32),
        pltpu.VMEM((B_PER_W,), jnp.int32),
        pltpu.VMEM((CHUNK, 4 * D), jnp.float32),
        pltpu.VMEM((CHUNK, 4 * D), jnp.float32),
        pltpu.VMEM((CHUNK // 4, 4 * D), jnp.float32),
        pltpu.VMEM((CHUNK // 4, 4 * D), jnp.float32),
        pltpu.SemaphoreType.DMA,
        pltpu.SemaphoreType.DMA,
        pltpu.SemaphoreType.DMA,
        pltpu.SemaphoreType.DMA,
    ],
    compiler_params=pltpu.CompilerParams(
        use_tc_tiling_on_sc=False, needs_layout_passes=False),
)
def _emb_gather(idx_hbm, table_hbm, out_hbm, colb_v, hi_v,
                rows0, rows1, ext0, ext1, sg0, sg1, so0, so1):
    wid = lax.axis_index("s") * NC + lax.axis_index("c")
    base = wid * B_PER_W
    pltpu.sync_copy(idx_hbm.at[pl.ds(base, B_PER_W)], colb_v)

    lane = lax.iota(jnp.int32, L)

    def prep(g, _):
        i = colb_v[pl.ds(g * L, L)]
        hi_v[pl.ds(g * L, L)] = lax.shift_right_logical(i, 2)
        colb_v[pl.ds(g * L, L)] = lax.shift_left(lax.bitwise_and(i, 3), 5)
        return _

    lax.fori_loop(0, NGRP, prep, 0)

    rows = (rows0, rows1)
    ext = (ext0, ext1)
    sg = (sg0, sg1)
    so = (so0, so1)

    exrow = [lax.shift_right_logical(lane * D + d, 7) for d in range(D)]
    excol = [lax.bitwise_and(lane * D + d, 127) for d in range(D)]

    def start_gather(c, u):
        return pltpu.async_copy(
            table_hbm.at[hi_v.at[pl.ds(c * CHUNK, CHUNK)]], rows[u], sg[u])

    def wait_gather(u):
        pltpu.make_async_copy(
            table_hbm.at[hi_v.at[pl.ds(0, CHUNK)]], rows[u], sg[u]).wait()

    def start_out(c, u):
        return pltpu.async_copy(
            ext[u], out_hbm.at[pl.ds(base // 4 + c * OUT_CH, OUT_CH)], so[u])

    def wait_out(u):
        pltpu.make_async_copy(
            ext[u], out_hbm.at[pl.ds(base // 4, OUT_CH)], so[u]).wait()

    def extract(c, u):
        def grp(g, _):
            cb = colb_v[pl.ds(c * CHUNK + g * L, L)]
            rvec = g * L + lane
            g4 = g * 4
            for d in range(D):
                x = plsc.load_gather(rows[u], [rvec, cb + d])
                plsc.store_scatter(ext[u], [g4 + exrow[d], excol[d]], x)
            return _

        lax.fori_loop(0, CHUNK // L, grp, 0)

    start_gather(0, 0)
    start_gather(1, 1)

    @pl.loop(0, NCHUNK - 2, step=2)
    def _ring(c0):
        for u in range(2):
            c = c0 + u
            wait_gather(u)

            @pl.when(c >= 2)
            def _():
                wait_out(u)

            extract(c, u)
            start_out(c, u)
            start_gather(c + 2, u)

    for u in range(2):
        c = NCHUNK - 2 + u
        wait_gather(u)
        wait_out(u)
        extract(c, u)
        start_out(c, u)
    wait_out(0)
    wait_out(1)


def kernel(X, W):
    idx = X.reshape(-1).astype(jnp.int32)
    table = W.reshape(250000, 128)
    out = _emb_gather(idx, table)
    return out.reshape(X.shape + (W.shape[1],))


# speedup vs baseline: 1.6646x; 1.6646x over previous
import functools

import jax
import jax.numpy as jnp
from jax import lax
from jax.experimental import pallas as pl
from jax.experimental.pallas import tpu as pltpu
from jax.experimental.pallas import tpu_sc as plsc

NC = 2
NS = 16
NW = NC * NS
L = 16

B, L1, L2 = 1024, 26, 20
B_TOTAL = B * L1 * L2
D = 32
B_PER_W = B_TOTAL // NW
BATCH_PER_W = B // NW
CB = 2
CHUNK = CB * L1 * L2
NCHUNK = BATCH_PER_W // CB
GPC = CHUNK // L


@functools.partial(
    pl.kernel,
    out_type=jax.ShapeDtypeStruct((B_TOTAL, D), jnp.float32),
    mesh=plsc.VectorSubcoreMesh(core_axis_name="c", subcore_axis_name="s"),
    scratch_types=[
        pltpu.VMEM((BATCH_PER_W, L1, L2), jnp.int32),
        pltpu.VMEM((B_PER_W,), jnp.int32),
        pltpu.VMEM((CHUNK, D), jnp.float32),
        pltpu.VMEM((CHUNK, D), jnp.float32),
        pltpu.SemaphoreType.DMA,
        pltpu.SemaphoreType.DMA,
        pltpu.SemaphoreType.DMA,
        pltpu.SemaphoreType.DMA,
    ],
    compiler_params=pltpu.CompilerParams(
        use_tc_tiling_on_sc=False, needs_layout_passes=False),
)
def _emb_gather(x_hbm, table_hbm, out_hbm, x_v, idx_v, rows0, rows1,
                sg0, sg1, so0, so1):
    wid = lax.axis_index("s") * NC + lax.axis_index("c")
    base = wid * B_PER_W
    pltpu.sync_copy(x_hbm.at[pl.ds(wid * BATCH_PER_W, BATCH_PER_W)], x_v)

    lane = lax.iota(jnp.int32, L)

    def flatten_chunk(c):
        def grp(g, carry):
            bv, iv, jv = carry
            x = plsc.load_gather(x_v, [bv, iv, jv])
            idx_v[pl.ds(c * CHUNK + g * L, L)] = x
            jn = jv + L
            mj = jn >= L2
            jn = jnp.where(mj, jn - L2, jn)
            iv = iv + mj.astype(jnp.int32)
            mi = iv >= L1
            iv = jnp.where(mi, iv - L1, iv)
            bv = bv + mi.astype(jnp.int32)
            return bv, iv, jn

        lax.fori_loop(
            0, GPC, grp,
            (jnp.full((L,), CB * c, jnp.int32), jnp.zeros((L,), jnp.int32),
             lane))

    rows = (rows0, rows1)
    sg = (sg0, sg1)
    so = (so0, so1)
    gathers = [None] * NCHUNK
    outs = [None] * NCHUNK
    for c in range(NCHUNK):
        b = c % 2
        if c >= 2:
            outs[c - 2].wait()
        flatten_chunk(c)
        gathers[c] = pltpu.async_copy(
            table_hbm.at[idx_v.at[pl.ds(c * CHUNK, CHUNK)]], rows[b], sg[b])
        if c >= 1:
            gathers[c - 1].wait()
            outs[c - 1] = pltpu.async_copy(
                rows[1 - b], out_hbm.at[pl.ds(base + (c - 1) * CHUNK, CHUNK)],
                so[1 - b])
    last = NCHUNK - 1
    gathers[last].wait()
    outs[last] = pltpu.async_copy(
        rows[last % 2], out_hbm.at[pl.ds(base + last * CHUNK, CHUNK)],
        so[last % 2])
    outs[last - 1].wait()
    outs[last].wait()


def kernel(X, W):
    out = _emb_gather(X, W)
    return out.reshape(X.shape + (W.shape[1],))
